# baseline (speedup 1.0000x reference)
import jax
import jax.numpy as jnp
from jax.experimental import pallas as pl
from jax.experimental.pallas import tpu as pltpu

_Q = 8192
_N = 16384
_K = 11
_BQ = 256


def _voronoi_block(points_ref, ct_ref, out_ref):
    p = points_ref[...]
    ct = ct_ref[...]
    p2 = jnp.sum(p * p, axis=1, keepdims=True)
    c2 = jnp.sum(ct * ct, axis=0, keepdims=True)
    mm = jax.lax.dot_general(
        p, ct, (((1,), (0,)), ((), ())),
        preferred_element_type=jnp.float32)
    d2 = (p2 + c2) - 2.0 * mm

    lane_iota = jax.lax.broadcasted_iota(jnp.int32, (_BQ, _N), 1)
    neg = -jnp.float32(jnp.inf)
    inf = jnp.float32(jnp.inf)

    m0 = jnp.min(d2, axis=1, keepdims=True)
    idx0 = jnp.min(jnp.where(d2 == m0, lane_iota, _N),
                   axis=1, keepdims=True)
    onehot0 = lane_iota == idx0
    c0x = jnp.max(jnp.where(onehot0, ct[0:1, :], neg), axis=1, keepdims=True)
    c0y = jnp.max(jnp.where(onehot0, ct[1:2, :], neg), axis=1, keepdims=True)
    c0z = jnp.max(jnp.where(onehot0, ct[2:3, :], neg), axis=1, keepdims=True)

    px = p[:, 0:1]
    py = p[:, 1:2]
    pz = p[:, 2:3]
    d0a = (px - c0x) ** 2 + (py - c0y) ** 2 + (pz - c0z) ** 2

    t = m0
    for _ in range(_K - 1):
        t = jnp.min(jnp.where(d2 > t, d2, inf), axis=1, keepdims=True)

    cx = ct[0:1, :]
    cy = ct[1:2, :]
    cz = ct[2:3, :]
    d2a = (px - cx) ** 2 + (py - cy) ** 2 + (pz - cz) ** 2
    ee = (c0x - cx) ** 2 + (c0y - cy) ** 2 + (c0z - cz) ** 2
    sq = (d2a - d0a) ** 2 * (0.25 * pl.reciprocal(ee, approx=True))
    sel = (d2 <= t) & (lane_iota != idx0)
    out_ref[...] = jnp.min(jnp.where(sel, sq, inf), axis=1, keepdims=True)


@jax.jit
def kernel(points, cell_points):
    ct = cell_points.T
    grid = _Q // _BQ
    out = pl.pallas_call(
        _voronoi_block,
        grid=(grid,),
        in_specs=[
            pl.BlockSpec((_BQ, 3), lambda i: (i, 0)),
            pl.BlockSpec((3, _N), lambda i: (0, 0)),
        ],
        out_specs=pl.BlockSpec((_BQ, 1), lambda i: (i, 0)),
        out_shape=jax.ShapeDtypeStruct((_Q, 1), jnp.float32),
        compiler_params=pltpu.CompilerParams(
            dimension_semantics=("parallel",),
        ),
    )(points, ct)
    return out.reshape(_Q)


# speedup vs baseline: 1.2118x; 1.2118x over previous
import jax
import jax.numpy as jnp
from jax.experimental import pallas as pl
from jax.experimental.pallas import tpu as pltpu

_Q = 8192
_N = 16384
_K = 11
_BQ = 256


def _voronoi_block(points_ref, ct_ref, out_ref):
    p = points_ref[...]
    ct = ct_ref[...]
    p2 = jnp.sum(p * p, axis=1, keepdims=True)
    c2 = jnp.sum(ct * ct, axis=0, keepdims=True)
    mm = jax.lax.dot_general(
        p, ct, (((1,), (0,)), ((), ())),
        preferred_element_type=jnp.float32)
    d2 = (p2 + c2) - 2.0 * mm

    lane_iota = jax.lax.broadcasted_iota(jnp.int32, (_BQ, _N), 1)
    neg = -jnp.float32(jnp.inf)
    inf = jnp.float32(jnp.inf)

    m0 = jnp.min(d2, axis=1, keepdims=True)
    idx0 = jnp.min(jnp.where(d2 == m0, lane_iota, _N),
                   axis=1, keepdims=True)
    onehot0 = lane_iota == idx0
    c0x = jnp.max(jnp.where(onehot0, ct[0:1, :], neg), axis=1, keepdims=True)
    c0y = jnp.max(jnp.where(onehot0, ct[1:2, :], neg), axis=1, keepdims=True)
    c0z = jnp.max(jnp.where(onehot0, ct[2:3, :], neg), axis=1, keepdims=True)

    px = p[:, 0:1]
    py = p[:, 1:2]
    pz = p[:, 2:3]
    d0a = (px - c0x) ** 2 + (py - c0y) ** 2 + (pz - c0z) ** 2

    def tbody(_, m):
        return jnp.min(jnp.where(d2 > m, d2, inf), axis=1, keepdims=True)

    t = jax.lax.fori_loop(0, _K - 1, tbody, m0)

    cx = ct[0:1, :]
    cy = ct[1:2, :]
    cz = ct[2:3, :]
    d2a = (px - cx) ** 2 + (py - cy) ** 2 + (pz - cz) ** 2
    ee = (c0x - cx) ** 2 + (c0y - cy) ** 2 + (c0z - cz) ** 2
    sq = (d2a - d0a) ** 2 * (0.25 * pl.reciprocal(ee, approx=True))
    sel = (d2 <= t) & (lane_iota != idx0)
    out_ref[...] = jnp.min(jnp.where(sel, sq, inf), axis=1, keepdims=True)


@jax.jit
def kernel(points, cell_points):
    ct = cell_points.T
    grid = _Q // _BQ
    out = pl.pallas_call(
        _voronoi_block,
        grid=(grid,),
        in_specs=[
            pl.BlockSpec((_BQ, 3), lambda i: (i, 0)),
            pl.BlockSpec((3, _N), lambda i: (0, 0)),
        ],
        out_specs=pl.BlockSpec((_BQ, 1), lambda i: (i, 0)),
        out_shape=jax.ShapeDtypeStruct((_Q, 1), jnp.float32),
        compiler_params=pltpu.CompilerParams(
            dimension_semantics=("parallel",),
        ),
    )(points, ct)
    return out.reshape(_Q)
